# baseline (speedup 1.0000x reference)
import jax
import jax.numpy as jnp
from jax import lax
from jax.experimental import pallas as pl
from jax.experimental.pallas import tpu as pltpu
from jax.experimental.pallas import tpu_sc as plsc

B, T, D = 16, 512, 256
MAX_LEN = 2048
L = 16
CHUNK = 128
ROWS_PER_W = B * MAX_LEN // 32
NCHUNK = ROWS_PER_W // CHUNK
NBUF = 3
ZROWS = 64


def _lr_body(x_hbm, dur_hbm, out_hbm,
             dur_v, cum_v, idx_v, dsti_v,
             rows_v0, rows_v1, rows_v2, zbuf,
             gsem0, gsem1, gsem2, osem0, osem1, osem2):
    cid = lax.axis_index("c")
    sid = lax.axis_index("s")
    b = sid
    r0 = ((cid + sid) % 2) * ROWS_PER_W

    bufs = (rows_v0, rows_v1, rows_v2)
    gsems = (gsem0, gsem1, gsem2)
    osems = (osem0, osem1, osem2)

    pltpu.sync_copy(dur_hbm.at[b], dur_v)

    zero_v = jnp.zeros((L,), jnp.float32)

    def cs_body(j, carry):
        v = jnp.maximum(dur_v[pl.ds(j * L, L)], 0)
        s = plsc.cumsum(v) + carry
        cum_v[pl.ds(j * L, L)] = s
        return s[L - 1]

    total = lax.fori_loop(0, T // L, cs_body, jnp.int32(0))
    limit = jnp.minimum(total, MAX_LEN)

    @pl.when(limit < r0 + ROWS_PER_W)
    def _zinit():
        def z_body(r, carry):
            for k in range(D // L):
                zbuf[r, pl.ds(k * L, L)] = zero_v
            return carry

        lax.fori_loop(0, ZROWS, z_body, 0)

    lane = lax.iota(jnp.int32, L)

    def compute_idx(c, permuted):
        def ss_body(j, _):
            if permuted:
                ofs = ((jnp.bitwise_and(lane, 7) * L + (lane >> 3))
                       + 2 * j)
            else:
                ofs = j * L + lane
            pos = r0 + c * CHUNK + ofs
            base = jnp.zeros((L,), jnp.int32)
            for half in (256, 128, 64, 32, 16, 8, 4, 2, 1):
                val = plsc.load_gather(cum_v, [base + (half - 1)])
                base = base + jnp.where(val <= pos, half, 0)
            val = plsc.load_gather(cum_v, [base])
            cnt = base + jnp.where(val <= pos, 1, 0)
            src = jnp.where(cnt > T - 1, jnp.bitwise_and(pos, T - 1), cnt)
            idx_v[pl.ds(c * CHUNK + j * L, L)] = b * T + src
            if permuted:
                dsti_v[c, pl.ds(j * L, L)] = out_base + c * CHUNK + ofs
            return 0

        lax.fori_loop(0, CHUNK // L, ss_body, 0)

    out_base = b * MAX_LEN + r0

    def finish_chunk(c):
        buf = bufs[c % NBUF]
        gpos0 = r0 + c * CHUNK
        live = gpos0 < limit
        full = gpos0 + CHUNK <= limit
        dst = out_hbm.at[pl.ds(out_base + c * CHUNK, CHUNK)]

        @pl.when(live)
        def _drain():
            pltpu.make_async_copy(
                x_hbm.at[idx_v.at[pl.ds(c * CHUNK, CHUNK)]],
                buf, gsems[c % NBUF]).wait()

        @pl.when(full)
        def _full():
            pltpu.async_copy(buf, out_hbm.at[dsti_v.at[c]], osems[c % NBUF])

        @pl.when(live & jnp.logical_not(full))
        def _boundary():
            mstart = jnp.clip(limit - gpos0, 0, CHUNK)

            def zrow(r, carry):
                for k in range(D // L):
                    buf[r, pl.ds(k * L, L)] = zero_v
                return carry

            lax.fori_loop(mstart, CHUNK, zrow, 0)
            pltpu.async_copy(buf, dst, osems[c % NBUF])

        @pl.when(jnp.logical_not(live))
        def _masked():
            pltpu.async_copy(zbuf, dst.at[pl.ds(0, ZROWS)], osems[c % NBUF])
            pltpu.async_copy(zbuf, dst.at[pl.ds(ZROWS, ZROWS)],
                             osems[c % NBUF])

    def drain_out(c):
        pltpu.make_async_copy(
            bufs[c % NBUF],
            out_hbm.at[pl.ds(out_base + c * CHUNK, CHUNK)],
            osems[c % NBUF]).wait()

    def issue_chunk(c):
        if c >= NBUF:
            drain_out(c - NBUF)
        gpos0 = r0 + c * CHUNK
        full = gpos0 + CHUNK <= limit

        @pl.when(full)
        def _idx_perm():
            compute_idx(c, permuted=True)

        @pl.when((gpos0 < limit) & jnp.logical_not(full))
        def _idx_ident():
            compute_idx(c, permuted=False)

        @pl.when(gpos0 < limit)
        def _issue():
            pltpu.async_copy(
                x_hbm.at[idx_v.at[pl.ds(c * CHUNK, CHUNK)]],
                bufs[c % NBUF], gsems[c % NBUF])

    issue_chunk(0)
    issue_chunk(1)
    for c in range(NCHUNK):
        if c + 2 < NCHUNK:
            issue_chunk(c + 2)
        finish_chunk(c)
    for c in range(NCHUNK - NBUF, NCHUNK):
        drain_out(c)


def kernel(x, durations, max_len):
    xflat = x.reshape(B * T, D)
    durflat = durations
    mesh = plsc.VectorSubcoreMesh(core_axis_name="c", subcore_axis_name="s",
                                  num_cores=2, num_subcores=16)
    run = pl.kernel(
        _lr_body,
        out_type=jax.ShapeDtypeStruct((B * MAX_LEN, D), jnp.float32),
        mesh=mesh,
        scratch_types=[
            pltpu.VMEM((T,), jnp.int32),
            pltpu.VMEM((T,), jnp.int32),
            pltpu.VMEM((ROWS_PER_W,), jnp.int32),
            pltpu.VMEM((NCHUNK, CHUNK), jnp.int32),
            pltpu.VMEM((CHUNK, D), jnp.float32),
            pltpu.VMEM((CHUNK, D), jnp.float32),
            pltpu.VMEM((CHUNK, D), jnp.float32),
            pltpu.VMEM((ZROWS, D), jnp.float32),
            pltpu.SemaphoreType.DMA,
            pltpu.SemaphoreType.DMA,
            pltpu.SemaphoreType.DMA,
            pltpu.SemaphoreType.DMA,
            pltpu.SemaphoreType.DMA,
            pltpu.SemaphoreType.DMA,
        ],
        compiler_params=pltpu.CompilerParams(needs_layout_passes=False),
    )
    out = run(xflat, durflat)
    return out.reshape(B, MAX_LEN, D)


# speedup vs baseline: 1.0019x; 1.0019x over previous
import jax
import jax.numpy as jnp
from jax import lax
from jax.experimental import pallas as pl
from jax.experimental.pallas import tpu as pltpu
from jax.experimental.pallas import tpu_sc as plsc

B, T, D = 16, 512, 256
MAX_LEN = 2048
L = 16
CHUNK = 64
ROWS_PER_W = B * MAX_LEN // 32
NCHUNK = ROWS_PER_W // CHUNK
NBUF = 6
AHEAD = 4
ZROWS = 64


def _lr_body(x_hbm, dur_hbm, out_hbm,
             dur_v, cum_v, idx_v,
             rows_v0, rows_v1, rows_v2, rows_v3, rows_v4, rows_v5, zbuf,
             gsem0, gsem1, gsem2, gsem3, gsem4, gsem5,
             osem0, osem1, osem2, osem3, osem4, osem5):
    cid = lax.axis_index("c")
    sid = lax.axis_index("s")
    b = sid
    r0 = ((cid + sid) % 2) * ROWS_PER_W

    bufs = (rows_v0, rows_v1, rows_v2, rows_v3, rows_v4, rows_v5)
    gsems = (gsem0, gsem1, gsem2, gsem3, gsem4, gsem5)
    osems = (osem0, osem1, osem2, osem3, osem4, osem5)

    pltpu.sync_copy(dur_hbm.at[b], dur_v)

    zero_v = jnp.zeros((L,), jnp.float32)

    def cs_body(j, carry):
        v = jnp.maximum(dur_v[pl.ds(j * L, L)], 0)
        s = plsc.cumsum(v) + carry
        cum_v[pl.ds(j * L, L)] = s
        return s[L - 1]

    total = lax.fori_loop(0, T // L, cs_body, jnp.int32(0))
    limit = jnp.minimum(total, MAX_LEN)

    @pl.when(limit < r0 + ROWS_PER_W)
    def _zinit():
        def z_body(r, carry):
            for k in range(D // L):
                zbuf[r, pl.ds(k * L, L)] = zero_v
            return carry

        lax.fori_loop(0, ZROWS, z_body, 0)

    lane = lax.iota(jnp.int32, L)

    def compute_idx(c):
        def ss_body(j, _):
            pos = r0 + c * CHUNK + j * L + lane
            base = jnp.zeros((L,), jnp.int32)
            for half in (256, 128, 64, 32, 16, 8, 4, 2, 1):
                val = plsc.load_gather(cum_v, [base + (half - 1)])
                base = base + jnp.where(val <= pos, half, 0)
            val = plsc.load_gather(cum_v, [base])
            cnt = base + jnp.where(val <= pos, 1, 0)
            src = jnp.where(cnt > T - 1, jnp.bitwise_and(pos, T - 1), cnt)
            idx_v[pl.ds(c * CHUNK + j * L, L)] = b * T + src
            return 0

        lax.fori_loop(0, CHUNK // L, ss_body, 0)

    out_base = b * MAX_LEN + r0

    def finish_chunk(c):
        buf = bufs[c % NBUF]
        gpos0 = r0 + c * CHUNK
        live = gpos0 < limit
        dst = out_hbm.at[pl.ds(out_base + c * CHUNK, CHUNK)]

        @pl.when(live)
        def _live():
            pltpu.make_async_copy(
                x_hbm.at[idx_v.at[pl.ds(c * CHUNK, CHUNK)]],
                buf, gsems[c % NBUF]).wait()
            mstart = jnp.clip(limit - gpos0, 0, CHUNK)

            def zrow(r, carry):
                for k in range(D // L):
                    buf[r, pl.ds(k * L, L)] = zero_v
                return carry

            lax.fori_loop(mstart, CHUNK, zrow, 0)
            pltpu.async_copy(buf, dst, osems[c % NBUF])

        @pl.when(jnp.logical_not(live))
        def _masked():
            pltpu.async_copy(zbuf, dst, osems[c % NBUF])

    def drain_out(c):
        pltpu.make_async_copy(
            bufs[c % NBUF],
            out_hbm.at[pl.ds(out_base + c * CHUNK, CHUNK)],
            osems[c % NBUF]).wait()

    def issue_chunk(c):
        if c >= NBUF:
            drain_out(c - NBUF)

        @pl.when(r0 + c * CHUNK < limit)
        def _issue():
            compute_idx(c)
            pltpu.async_copy(
                x_hbm.at[idx_v.at[pl.ds(c * CHUNK, CHUNK)]],
                bufs[c % NBUF], gsems[c % NBUF])

    for c in range(AHEAD):
        issue_chunk(c)
    for c in range(NCHUNK):
        if c + AHEAD < NCHUNK:
            issue_chunk(c + AHEAD)
        finish_chunk(c)
    for c in range(NCHUNK - NBUF, NCHUNK):
        drain_out(c)


def kernel(x, durations, max_len):
    xflat = x.reshape(B * T, D)
    durflat = durations
    mesh = plsc.VectorSubcoreMesh(core_axis_name="c", subcore_axis_name="s",
                                  num_cores=2, num_subcores=16)
    run = pl.kernel(
        _lr_body,
        out_type=jax.ShapeDtypeStruct((B * MAX_LEN, D), jnp.float32),
        mesh=mesh,
        scratch_types=[
            pltpu.VMEM((T,), jnp.int32),
            pltpu.VMEM((T,), jnp.int32),
            pltpu.VMEM((ROWS_PER_W,), jnp.int32),
            pltpu.VMEM((CHUNK, D), jnp.float32),
            pltpu.VMEM((CHUNK, D), jnp.float32),
            pltpu.VMEM((CHUNK, D), jnp.float32),
            pltpu.VMEM((CHUNK, D), jnp.float32),
            pltpu.VMEM((CHUNK, D), jnp.float32),
            pltpu.VMEM((CHUNK, D), jnp.float32),
            pltpu.VMEM((ZROWS, D), jnp.float32),
            pltpu.SemaphoreType.DMA,
            pltpu.SemaphoreType.DMA,
            pltpu.SemaphoreType.DMA,
            pltpu.SemaphoreType.DMA,
            pltpu.SemaphoreType.DMA,
            pltpu.SemaphoreType.DMA,
            pltpu.SemaphoreType.DMA,
            pltpu.SemaphoreType.DMA,
            pltpu.SemaphoreType.DMA,
            pltpu.SemaphoreType.DMA,
            pltpu.SemaphoreType.DMA,
            pltpu.SemaphoreType.DMA,

        ],
        compiler_params=pltpu.CompilerParams(needs_layout_passes=False),
    )
    out = run(xflat, durflat)
    return out.reshape(B, MAX_LEN, D)


# speedup vs baseline: 1.0347x; 1.0328x over previous
import jax
import jax.numpy as jnp
from jax import lax
from jax.experimental import pallas as pl
from jax.experimental.pallas import tpu as pltpu
from jax.experimental.pallas import tpu_sc as plsc

B, T, D = 16, 512, 256
MAX_LEN = 2048
L = 16
CHUNK = 128
ROWS_PER_W = B * MAX_LEN // 32
NCHUNK = ROWS_PER_W // CHUNK
NBUF = 3
ZROWS = 64


def _lr_body(x_hbm, dur_hbm, out_hbm,
             dur_v, cum_v, idx_v,
             rows_v0, rows_v1, rows_v2, zbuf,
             gsem0, gsem1, gsem2, osem0, osem1, osem2):
    cid = lax.axis_index("c")
    sid = lax.axis_index("s")
    b = sid
    r0 = ((cid + sid) % 2) * ROWS_PER_W

    bufs = (rows_v0, rows_v1, rows_v2)
    gsems = (gsem0, gsem1, gsem2)
    osems = (osem0, osem1, osem2)

    pltpu.sync_copy(dur_hbm.at[b], dur_v)

    zero_v = jnp.zeros((L,), jnp.float32)

    def cs_body(j, carry):
        v = jnp.maximum(dur_v[pl.ds(j * L, L)], 0)
        s = plsc.cumsum(v) + carry
        cum_v[pl.ds(j * L, L)] = s
        return s[L - 1]

    total = lax.fori_loop(0, T // L, cs_body, jnp.int32(0))
    limit = jnp.minimum(total, MAX_LEN)

    @pl.when(limit < r0 + ROWS_PER_W)
    def _zinit():
        def z_body(r, carry):
            for k in range(D // L):
                zbuf[r, pl.ds(k * L, L)] = zero_v
            return carry

        lax.fori_loop(0, ZROWS, z_body, 0)

    lane = lax.iota(jnp.int32, L)

    def compute_idx(c):
        def ss_body(j, _):
            pos = r0 + c * CHUNK + j * L + lane
            base = jnp.zeros((L,), jnp.int32)
            for half in (256, 128, 64, 32, 16, 8, 4, 2, 1):
                val = plsc.load_gather(cum_v, [base + (half - 1)])
                base = base + jnp.where(val <= pos, half, 0)
            val = plsc.load_gather(cum_v, [base])
            cnt = base + jnp.where(val <= pos, 1, 0)
            src = jnp.where(cnt > T - 1, jnp.bitwise_and(pos, T - 1), cnt)
            idx_v[pl.ds(c * CHUNK + j * L, L)] = b * T + src
            return 0

        lax.fori_loop(0, CHUNK // L, ss_body, 0)

    out_base = b * MAX_LEN + r0

    def finish_chunk(c):
        buf = bufs[c % NBUF]
        gpos0 = r0 + c * CHUNK
        live = gpos0 < limit
        dst = out_hbm.at[pl.ds(out_base + c * CHUNK, CHUNK)]

        @pl.when(live)
        def _live():
            pltpu.make_async_copy(
                x_hbm.at[idx_v.at[pl.ds(c * CHUNK, CHUNK)]],
                buf, gsems[c % NBUF]).wait()
            mstart = jnp.clip(limit - gpos0, 0, CHUNK)

            def zrow(r, carry):
                for k in range(D // L):
                    buf[r, pl.ds(k * L, L)] = zero_v
                return carry

            lax.fori_loop(mstart, CHUNK, zrow, 0)
            pltpu.async_copy(buf, dst, osems[c % NBUF])

        @pl.when(jnp.logical_not(live))
        def _masked():
            pltpu.async_copy(zbuf, dst.at[pl.ds(0, ZROWS)], osems[c % NBUF])
            pltpu.async_copy(zbuf, dst.at[pl.ds(ZROWS, ZROWS)],
                             osems[c % NBUF])

    def drain_out(c):
        pltpu.make_async_copy(
            bufs[c % NBUF],
            out_hbm.at[pl.ds(out_base + c * CHUNK, CHUNK)],
            osems[c % NBUF]).wait()

    def issue_chunk(c):
        if c >= NBUF:
            drain_out(c - NBUF)

        @pl.when(r0 + c * CHUNK < limit)
        def _issue():
            compute_idx(c)
            pltpu.async_copy(
                x_hbm.at[idx_v.at[pl.ds(c * CHUNK, CHUNK)]],
                bufs[c % NBUF], gsems[c % NBUF])

    issue_chunk(0)
    issue_chunk(1)
    for c in range(NCHUNK):
        if c + 2 < NCHUNK:
            issue_chunk(c + 2)
        finish_chunk(c)
    for c in range(NCHUNK - NBUF, NCHUNK):
        drain_out(c)


def kernel(x, durations, max_len):
    xflat = x.reshape(B * T, D)
    durflat = durations
    mesh = plsc.VectorSubcoreMesh(core_axis_name="c", subcore_axis_name="s",
                                  num_cores=2, num_subcores=16)
    run = pl.kernel(
        _lr_body,
        out_type=jax.ShapeDtypeStruct((B * MAX_LEN, D), jnp.float32),
        mesh=mesh,
        scratch_types=[
            pltpu.VMEM((T,), jnp.int32),
            pltpu.VMEM((T,), jnp.int32),
            pltpu.VMEM((ROWS_PER_W,), jnp.int32),
            pltpu.VMEM((CHUNK, D), jnp.float32),
            pltpu.VMEM((CHUNK, D), jnp.float32),
            pltpu.VMEM((CHUNK, D), jnp.float32),
            pltpu.VMEM((ZROWS, D), jnp.float32),
            pltpu.SemaphoreType.DMA,
            pltpu.SemaphoreType.DMA,
            pltpu.SemaphoreType.DMA,
            pltpu.SemaphoreType.DMA,
            pltpu.SemaphoreType.DMA,
            pltpu.SemaphoreType.DMA,
        ],
        compiler_params=pltpu.CompilerParams(needs_layout_passes=False),
    )
    out = run(xflat, durflat)
    return out.reshape(B, MAX_LEN, D)
